# Initial kernel scaffold; baseline (speedup 1.0000x reference)
import functools

import jax
import jax.numpy as jnp
from jax import lax
from jax.experimental import pallas as pl
from jax.experimental.pallas import tpu as pltpu
from jax.experimental.pallas import tpu_sc as plsc

_EPS = 1e-12
_GATHER_WINDOW = 64


def _sc_gather(word_emb, flat_ids):
    n_tok = flat_ids.shape[0]
    emb = word_emb.shape[1]
    w = _GATHER_WINDOW
    mesh = plsc.VectorSubcoreMesh(core_axis_name="c", subcore_axis_name="s")

    @functools.partial(
        pl.kernel,
        out_type=jax.ShapeDtypeStruct((n_tok, emb), jnp.float32),
        mesh=mesh,
    )
    def gather_kernel(tab_hbm, idx_hbm, o_hbm):
        def body(i_vmem, o_vmem):
            pltpu.sync_copy(tab_hbm.at[i_vmem.at[0]], o_vmem)

        pltpu.emit_pipeline(
            body,
            grid=(n_tok // w,),
            in_specs=[pl.BlockSpec((1, w), lambda i: (0, i))],
            out_specs=[pl.BlockSpec((w, emb), lambda i: (i, 0))],
            core_axis_name=("c", "s"),
            dimension_semantics=(pltpu.PARALLEL,),
        )(idx_hbm, o_hbm)

    return gather_kernel(word_emb, flat_ids.reshape(1, n_tok))


def _tc_add_ln(gathered, token_type_ids, pe, type_emb, gamma, beta, nb=32):
    b, s = token_type_ids.shape
    e = gathered.shape[-1]
    g3 = gathered.reshape(b, s, e)

    def body(g_ref, tti_ref, pe_ref, te_ref, gam_ref, bet_ref, o_ref):
        g = g_ref[...]
        tti = tti_ref[...]
        t0 = te_ref[0, :]
        t1 = te_ref[1, :]
        tte = jnp.where((tti == 0)[..., None], t0[None, None, :], t1[None, None, :])
        x = g + pe_ref[...][None, :, :] + tte
        mu = jnp.mean(x, axis=-1, keepdims=True)
        xc = x - mu
        var = jnp.mean(xc * xc, axis=-1, keepdims=True)
        y = xc * lax.rsqrt(var + _EPS)
        o_ref[...] = y * gam_ref[...] + bet_ref[...]

    return pl.pallas_call(
        body,
        grid=(b // nb,),
        in_specs=[
            pl.BlockSpec((nb, s, e), lambda i: (i, 0, 0)),
            pl.BlockSpec((nb, s), lambda i: (i, 0)),
            pl.BlockSpec((s, e), lambda i: (0, 0)),
            pl.BlockSpec((2, e), lambda i: (0, 0)),
            pl.BlockSpec((e,), lambda i: (0,)),
            pl.BlockSpec((e,), lambda i: (0,)),
        ],
        out_specs=pl.BlockSpec((nb, s, e), lambda i: (i, 0, 0)),
        out_shape=jax.ShapeDtypeStruct((b, s, e), jnp.float32),
    )(g3, token_type_ids, pe, type_emb, gamma, beta)


def kernel(token_ids, token_type_ids, word_emb, pos_emb, type_emb, ln_gamma, ln_beta):
    b, s = token_ids.shape
    e = word_emb.shape[1]
    flat_ids = token_ids.reshape(b * s).astype(jnp.int32)
    gathered = _sc_gather(word_emb, flat_ids)
    pe = lax.slice(pos_emb, (0, 0), (s, e))
    return _tc_add_ln(
        gathered,
        token_type_ids.astype(jnp.int32),
        pe,
        type_emb,
        ln_gamma,
        ln_beta,
    )


# speedup vs baseline: 1.9669x; 1.9669x over previous
import functools

import jax
import jax.numpy as jnp
from jax import lax
from jax.experimental import pallas as pl
from jax.experimental.pallas import tpu as pltpu
from jax.experimental.pallas import tpu_sc as plsc

_EPS = 1e-12
_GATHER_WINDOW = 160


def _sc_gather(word_emb, flat_ids):
    n_tok = flat_ids.shape[0]
    emb = word_emb.shape[1]
    nc, ns = 2, 16
    nw = nc * ns
    n_per_w = n_tok // nw
    chunk = _GATHER_WINDOW
    n_chunks = n_per_w // chunk
    mesh = plsc.VectorSubcoreMesh(core_axis_name="c", subcore_axis_name="s")

    @functools.partial(
        pl.kernel,
        out_type=jax.ShapeDtypeStruct((n_tok, emb), jnp.float32),
        mesh=mesh,
        scratch_types=[
            pltpu.VMEM((n_per_w,), jnp.int32),
            pltpu.VMEM((chunk, emb), jnp.float32),
            pltpu.SemaphoreType.DMA,
        ],
    )
    def gather_kernel(tab_hbm, idx_hbm, o_hbm, idx_v, rows_v, sem):
        wid = lax.axis_index("s") * nc + lax.axis_index("c")
        base = wid * n_per_w
        pltpu.sync_copy(idx_hbm.at[pl.ds(base, n_per_w)], idx_v)

        @pl.loop(0, n_chunks)
        def _(j):
            off = j * chunk
            pltpu.async_copy(
                tab_hbm.at[idx_v.at[pl.ds(off, chunk)]], rows_v, sem
            ).wait()
            pltpu.sync_copy(rows_v, o_hbm.at[pl.ds(base + off, chunk)])

    return gather_kernel(word_emb, flat_ids)


def _tc_add_ln(gathered, tti_f, pe, type_emb, gamma, beta, nb=32):
    b, s = tti_f.shape[:2]
    e = gathered.shape[-1]
    g3 = gathered.reshape(b, s, e)

    def body(g_ref, tti_ref, pe_ref, te_ref, gam_ref, bet_ref, o_ref):
        g = g_ref[...]
        t0 = te_ref[0, :]
        t1 = te_ref[1, :]
        tte = t0 + tti_ref[...] * (t1 - t0)
        x = g + pe_ref[...] + tte
        mu = jnp.mean(x, axis=-1, keepdims=True)
        xc = x - mu
        var = jnp.mean(xc * xc, axis=-1, keepdims=True)
        y = xc * lax.rsqrt(var + _EPS)
        o_ref[...] = y * gam_ref[...] + bet_ref[...]

    return pl.pallas_call(
        body,
        grid=(b // nb,),
        in_specs=[
            pl.BlockSpec((nb, s, e), lambda i: (i, 0, 0)),
            pl.BlockSpec((nb, s, 1), lambda i: (i, 0, 0)),
            pl.BlockSpec((s, e), lambda i: (0, 0)),
            pl.BlockSpec((2, e), lambda i: (0, 0)),
            pl.BlockSpec((e,), lambda i: (0,)),
            pl.BlockSpec((e,), lambda i: (0,)),
        ],
        out_specs=pl.BlockSpec((nb, s, e), lambda i: (i, 0, 0)),
        out_shape=jax.ShapeDtypeStruct((b, s, e), jnp.float32),
    )(g3, tti_f, pe, type_emb, gamma, beta)


def kernel(token_ids, token_type_ids, word_emb, pos_emb, type_emb, ln_gamma, ln_beta):
    b, s = token_ids.shape
    e = word_emb.shape[1]
    flat_ids = token_ids.reshape(b * s).astype(jnp.int32)
    gathered = _sc_gather(word_emb, flat_ids)
    pe = lax.slice(pos_emb, (0, 0), (s, e))
    tti_f = token_type_ids[..., None].astype(jnp.float32)
    return _tc_add_ln(gathered, tti_f, pe, type_emb, ln_gamma, ln_beta)
